# baseline (speedup 1.0000x reference)
import functools

import jax
import jax.numpy as jnp
from jax import lax
from jax.experimental import pallas as pl
from jax.experimental.pallas import tpu as pltpu
from jax.experimental.pallas import tpu_sc as plsc

IMG_H, IMG_W = 512, 512
PATCH_H, PATCH_W = 64, 64
PATCH_NUM = 16
B, C = 4, 96

NUM_CORES = 2
NUM_SUBCORES = 16
NUM_WORKERS = NUM_CORES * NUM_SUBCORES
C_PER_WORKER = C // NUM_WORKERS
ITEMS = PATCH_NUM * B * C_PER_WORKER

N_ROWS = B * C * IMG_H
BLK_H = PATCH_H + 8
BLK_W = 256


def _patch_body(x_hbm, ys_hbm, xs_hbm, out_hbm,
                ysv, xsv, ysm, xsm,
                buf0, buf1, ob0, ob1, si0, si1, so0, so1):
    wid = lax.axis_index("s") * NUM_CORES + lax.axis_index("c")
    pltpu.sync_copy(ys_hbm, ysv)
    pltpu.sync_copy(xs_hbm, xsv)
    lane = lax.iota(jnp.int32, 16)

    def stash_body(p, carry):
        mask = lane == p
        ysm[p] = jnp.sum(jnp.where(mask, ysv[...], 0))
        xsm[p] = jnp.sum(jnp.where(mask, xsv[...], 0))
        return carry

    lax.fori_loop(0, PATCH_NUM, stash_body, 0)

    bufs = (buf0, buf1)
    obs = (ob0, ob1)
    sins = (si0, si1)
    souts = (so0, so1)

    def item_coords(k):
        p = k & 15
        i = k >> 4
        j = i >> 2
        b = i & 3
        y0 = ysm[p]
        x0 = xsm[p]
        c = wid * C_PER_WORKER + j
        grow = (b * C + c) * IMG_H + y0
        ra = jnp.minimum(grow & ~7, N_ROWS - BLK_H)
        dy = grow - ra
        xa = jnp.minimum(x0 & ~127, IMG_W - BLK_W)
        dx = x0 - xa
        return ra, xa, dy, dx, p * B + b, c

    def start_in(k, par):
        ra, xa, _, _, _, _ = item_coords(k)
        src = x_hbm.at[
            pl.ds(pl.multiple_of(ra, 8), BLK_H),
            pl.ds(pl.multiple_of(xa, 128), BLK_W),
        ]
        pltpu.async_copy(src, bufs[par], sins[par])

    def wait_in(par):
        pltpu.make_async_copy(
            x_hbm.at[pl.ds(0, BLK_H), pl.ds(0, BLK_W)], bufs[par], sins[par]
        ).wait()

    def start_out(pb, c, par):
        pltpu.async_copy(obs[par], out_hbm.at[pb, c], souts[par])

    def wait_out(par):
        pltpu.make_async_copy(obs[par], out_hbm.at[0, 0], souts[par]).wait()

    def extract(dy, dx, par):
        buf = bufs[par]
        ob = obs[par]
        colvs = [dx + jj * 16 + lane for jj in range(PATCH_W // 16)]

        def r_body(r, _):
            rowv = lax.full((16,), 0, jnp.int32) + (dy + r)
            for jj in range(PATCH_W // 16):
                v = plsc.load_gather(buf, [rowv, colvs[jj]])
                ob[r, pl.ds(jj * 16, 16)] = v
            return _

        lax.fori_loop(0, PATCH_H, r_body, 0)

    start_in(0, 0)
    start_in(1, 1)

    def pair_body(k2, carry):
        for par in (0, 1):
            k = 2 * k2 + par
            _, _, dy, dx, pb, c = item_coords(k)
            wait_in(par)

            @pl.when(k >= 2)
            def _():
                wait_out(par)

            extract(dy, dx, par)
            start_out(pb, c, par)

            @pl.when(k + 2 < ITEMS)
            def _():
                start_in(k + 2, par)
        return carry

    lax.fori_loop(0, ITEMS // 2, pair_body, 0)
    wait_out(0)
    wait_out(1)


@jax.jit
def _run(x2, ys, xs):
    mesh = plsc.VectorSubcoreMesh(core_axis_name="c", subcore_axis_name="s")
    f = functools.partial(
        pl.kernel,
        mesh=mesh,
        out_type=jax.ShapeDtypeStruct(
            (PATCH_NUM * B, C, PATCH_H, PATCH_W), jnp.float32
        ),
        scratch_types=[
            pltpu.VMEM((16,), jnp.int32),
            pltpu.VMEM((16,), jnp.int32),
            pltpu.SMEM((PATCH_NUM,), jnp.int32),
            pltpu.SMEM((PATCH_NUM,), jnp.int32),
            pltpu.VMEM((BLK_H, BLK_W), jnp.float32),
            pltpu.VMEM((BLK_H, BLK_W), jnp.float32),
            pltpu.VMEM((PATCH_H, PATCH_W), jnp.float32),
            pltpu.VMEM((PATCH_H, PATCH_W), jnp.float32),
            pltpu.SemaphoreType.DMA,
            pltpu.SemaphoreType.DMA,
            pltpu.SemaphoreType.DMA,
            pltpu.SemaphoreType.DMA,
        ],
        compiler_params=pltpu.CompilerParams(needs_layout_passes=False),
    )(_patch_body)
    return f(x2, ys, xs)


def kernel(input, patch_indices):
    x2 = input.reshape(B * C * IMG_H, IMG_W)
    pidx = patch_indices.astype(jnp.int32)
    ys = pidx[:, 0]
    xs = pidx[:, 1]
    return _run(x2, ys, xs)


# speedup vs baseline: 1.0002x; 1.0002x over previous
import functools

import jax
import jax.numpy as jnp
from jax import lax
from jax.experimental import pallas as pl
from jax.experimental.pallas import tpu as pltpu
from jax.experimental.pallas import tpu_sc as plsc

IMG_H, IMG_W = 512, 512
PATCH_H, PATCH_W = 64, 64
PATCH_NUM = 16
B, C = 4, 96

NUM_CORES = 2
NUM_SUBCORES = 16
NUM_WORKERS = NUM_CORES * NUM_SUBCORES
C_PER_WORKER = C // NUM_WORKERS
ITEMS = PATCH_NUM * B * C_PER_WORKER

N_ROWS = B * C * IMG_H
BLK_H = PATCH_H + 8
BLK_W = 256


def _patch_body(x_hbm, ys_hbm, xs_hbm, out_hbm,
                ysv, xsv, ysm, xsm,
                buf0, buf1, ob0, ob1, si0, si1, so0, so1):
    wid = lax.axis_index("s") * NUM_CORES + lax.axis_index("c")
    pltpu.sync_copy(ys_hbm, ysv)
    pltpu.sync_copy(xs_hbm, xsv)
    lane = lax.iota(jnp.int32, 16)

    def stash_body(p, carry):
        mask = lane == p
        ysm[p] = jnp.sum(jnp.where(mask, ysv[...], 0))
        xsm[p] = jnp.sum(jnp.where(mask, xsv[...], 0))
        return carry

    lax.fori_loop(0, PATCH_NUM, stash_body, 0)

    bufs = (buf0, buf1)
    obs = (ob0, ob1)
    sins = (si0, si1)
    souts = (so0, so1)

    def item_coords(k):
        p = k & 15
        i = k >> 4
        j = i >> 2
        b = i & 3
        mask = lane == p
        y0 = jnp.sum(jnp.where(mask, ysv[...], 0))
        x0 = jnp.sum(jnp.where(mask, xsv[...], 0))
        c = wid * C_PER_WORKER + j
        grow = (b * C + c) * IMG_H + y0
        ra = jnp.minimum(grow & ~7, N_ROWS - BLK_H)
        dy = grow - ra
        xa = jnp.minimum(x0 & ~127, IMG_W - BLK_W)
        dx = x0 - xa
        return ra, xa, dy, dx, p * B + b, c

    def start_in(k, par):
        ra, xa, _, _, _, _ = item_coords(k)
        src = x_hbm.at[
            pl.ds(pl.multiple_of(ra, 8), BLK_H),
            pl.ds(pl.multiple_of(xa, 128), BLK_W),
        ]
        pltpu.async_copy(src, bufs[par], sins[par])

    def wait_in(par):
        pltpu.make_async_copy(
            x_hbm.at[pl.ds(0, BLK_H), pl.ds(0, BLK_W)], bufs[par], sins[par]
        ).wait()

    def start_out(pb, c, par):
        pltpu.async_copy(obs[par], out_hbm.at[pb, c], souts[par])

    def wait_out(par):
        pltpu.make_async_copy(obs[par], out_hbm.at[0, 0], souts[par]).wait()

    def extract(dy, dx, par):
        buf = bufs[par]
        ob = obs[par]
        colvs = [dx + jj * 16 + lane for jj in range(PATCH_W // 16)]

        def r_body(r, _):
            rowv = lax.full((16,), 0, jnp.int32) + (dy + r)
            for jj in range(PATCH_W // 16):
                v = plsc.load_gather(buf, [rowv, colvs[jj]])
                ob[r, pl.ds(jj * 16, 16)] = v
            return _

        lax.fori_loop(0, PATCH_H, r_body, 0)

    start_in(0, 0)
    start_in(1, 1)

    def pair_body(k2, carry):
        for par in (0, 1):
            k = 2 * k2 + par
            _, _, dy, dx, pb, c = item_coords(k)
            wait_in(par)

            @pl.when(k >= 2)
            def _():
                wait_out(par)

            extract(dy, dx, par)
            start_out(pb, c, par)

            @pl.when(k + 2 < ITEMS)
            def _():
                start_in(k + 2, par)
        return carry

    lax.fori_loop(0, ITEMS // 2, pair_body, 0)
    wait_out(0)
    wait_out(1)


@jax.jit
def _run(x2, ys, xs):
    mesh = plsc.VectorSubcoreMesh(core_axis_name="c", subcore_axis_name="s")
    f = functools.partial(
        pl.kernel,
        mesh=mesh,
        out_type=jax.ShapeDtypeStruct(
            (PATCH_NUM * B, C, PATCH_H, PATCH_W), jnp.float32
        ),
        scratch_types=[
            pltpu.VMEM((16,), jnp.int32),
            pltpu.VMEM((16,), jnp.int32),
            pltpu.SMEM((PATCH_NUM,), jnp.int32),
            pltpu.SMEM((PATCH_NUM,), jnp.int32),
            pltpu.VMEM((BLK_H, BLK_W), jnp.float32),
            pltpu.VMEM((BLK_H, BLK_W), jnp.float32),
            pltpu.VMEM((PATCH_H, PATCH_W), jnp.float32),
            pltpu.VMEM((PATCH_H, PATCH_W), jnp.float32),
            pltpu.SemaphoreType.DMA,
            pltpu.SemaphoreType.DMA,
            pltpu.SemaphoreType.DMA,
            pltpu.SemaphoreType.DMA,
        ],
        compiler_params=pltpu.CompilerParams(needs_layout_passes=False),
    )(_patch_body)
    return f(x2, ys, xs)


def kernel(input, patch_indices):
    x2 = input.reshape(B * C * IMG_H, IMG_W)
    pidx = patch_indices.astype(jnp.int32)
    ys = pidx[:, 0]
    xs = pidx[:, 1]
    return _run(x2, ys, xs)


# speedup vs baseline: 1.3474x; 1.3470x over previous
import functools

import jax
import jax.numpy as jnp
from jax import lax
from jax.experimental import pallas as pl
from jax.experimental.pallas import tpu as pltpu
from jax.experimental.pallas import tpu_sc as plsc

IMG_H, IMG_W = 512, 512
PATCH_H, PATCH_W = 64, 64
PATCH_NUM = 16
B, C = 4, 96

NUM_CORES = 2
NUM_SUBCORES = 16
NUM_WORKERS = NUM_CORES * NUM_SUBCORES
C_PER_WORKER = C // NUM_WORKERS
ITEMS = PATCH_NUM * B * C_PER_WORKER
NBUF = 4

N_ROWS = B * C * IMG_H
BLK_H = PATCH_H + 8
BLK_W = 256


def _patch_body(x_hbm, ys_hbm, xs_hbm, out_hbm, ysv, xsv, *bufs_obs_sems):
    bufs = bufs_obs_sems[0:NBUF]
    obs = bufs_obs_sems[NBUF:2 * NBUF]
    sins = bufs_obs_sems[2 * NBUF:3 * NBUF]
    souts = bufs_obs_sems[3 * NBUF:4 * NBUF]

    wid = lax.axis_index("s") * NUM_CORES + lax.axis_index("c")
    pltpu.sync_copy(ys_hbm, ysv)
    pltpu.sync_copy(xs_hbm, xsv)
    lane = lax.iota(jnp.int32, 16)

    def item_coords(k):
        p = k & 15
        i = k >> 4
        j = i >> 2
        b = i & 3
        mask = lane == p
        y0 = jnp.sum(jnp.where(mask, ysv[...], 0))
        x0 = jnp.sum(jnp.where(mask, xsv[...], 0))
        c = wid * C_PER_WORKER + j
        grow = (b * C + c) * IMG_H + y0
        ra = jnp.minimum(grow & ~7, N_ROWS - BLK_H)
        dy = grow - ra
        xa = jnp.minimum(x0 & ~127, IMG_W - BLK_W)
        dx = x0 - xa
        orow = ((p * B + b) * C + c) * PATCH_H
        return ra, xa, dy, dx, orow

    def start_in(k, par):
        ra, xa, _, _, _ = item_coords(k)
        src = x_hbm.at[
            pl.ds(pl.multiple_of(ra, 8), BLK_H),
            pl.ds(pl.multiple_of(xa, 128), BLK_W),
        ]
        pltpu.async_copy(src, bufs[par], sins[par])

    def wait_in(par):
        pltpu.make_async_copy(
            x_hbm.at[pl.ds(0, BLK_H), pl.ds(0, BLK_W)], bufs[par], sins[par]
        ).wait()

    def start_out(orow, par):
        dst = out_hbm.at[pl.ds(pl.multiple_of(orow, 8), PATCH_H), :]
        pltpu.async_copy(obs[par], dst, souts[par])

    def wait_out(par):
        pltpu.make_async_copy(
            obs[par], out_hbm.at[pl.ds(0, PATCH_H), :], souts[par]
        ).wait()

    def extract(dy, dx, par):
        buf = bufs[par]
        ob = obs[par]
        colvs = [dx + jj * 16 + lane for jj in range(PATCH_W // 16)]

        def r_body(r, _):
            rowv = lax.full((16,), 0, jnp.int32) + (dy + r)
            for jj in range(PATCH_W // 16):
                v = plsc.load_gather(buf, [rowv, colvs[jj]])
                ob[r, pl.ds(jj * 16, 16)] = v
            return _

        lax.fori_loop(0, PATCH_H, r_body, 0)

    for par in range(NBUF):
        start_in(par, par)

    def group_body(kg, carry):
        for par in range(NBUF):
            k = NBUF * kg + par
            _, _, dy, dx, orow = item_coords(k)
            wait_in(par)

            @pl.when(k >= NBUF)
            def _():
                wait_out(par)

            extract(dy, dx, par)
            start_out(orow, par)

            @pl.when(k + NBUF < ITEMS)
            def _():
                start_in(k + NBUF, par)
        return carry

    lax.fori_loop(0, ITEMS // NBUF, group_body, 0)
    for par in range(NBUF):
        wait_out(par)


@jax.jit
def _run(x2, ys, xs):
    mesh = plsc.VectorSubcoreMesh(core_axis_name="c", subcore_axis_name="s")
    f = functools.partial(
        pl.kernel,
        mesh=mesh,
        out_type=jax.ShapeDtypeStruct(
            (PATCH_NUM * B * C * PATCH_H, PATCH_W), jnp.float32
        ),
        scratch_types=(
            [
                pltpu.VMEM((16,), jnp.int32),
                pltpu.VMEM((16,), jnp.int32),
            ]
            + [pltpu.VMEM((BLK_H, BLK_W), jnp.float32)] * NBUF
            + [pltpu.VMEM((PATCH_H, PATCH_W), jnp.float32)] * NBUF
            + [pltpu.SemaphoreType.DMA] * (2 * NBUF)
        ),
        compiler_params=pltpu.CompilerParams(needs_layout_passes=False),
    )(_patch_body)
    return f(x2, ys, xs)


def kernel(input, patch_indices):
    x2 = input.reshape(B * C * IMG_H, IMG_W)
    pidx = patch_indices.astype(jnp.int32)
    ys = pidx[:, 0]
    xs = pidx[:, 1]
    out2 = _run(x2, ys, xs)
    return out2.reshape(PATCH_NUM * B, C, PATCH_H, PATCH_W)


# speedup vs baseline: 1.3584x; 1.0082x over previous
import functools

import jax
import jax.numpy as jnp
from jax import lax
from jax.experimental import pallas as pl
from jax.experimental.pallas import tpu as pltpu
from jax.experimental.pallas import tpu_sc as plsc

IMG_H, IMG_W = 512, 512
PATCH_H, PATCH_W = 64, 64
PATCH_NUM = 16
B, C = 4, 96

NUM_CORES = 2
NUM_SUBCORES = 16
NUM_WORKERS = NUM_CORES * NUM_SUBCORES
C_PER_WORKER = C // NUM_WORKERS
ITEMS = PATCH_NUM * B * C_PER_WORKER
NBUF = 4

N_ROWS = B * C * IMG_H
BLK_H = PATCH_H + 8
BLK_W = 256


def _patch_body(x_hbm, ys_hbm, xs_hbm, out_hbm, ysv, xsv, *bufs_obs_sems):
    bufs = bufs_obs_sems[0:NBUF]
    obs = bufs_obs_sems[NBUF:2 * NBUF]
    sins = bufs_obs_sems[2 * NBUF:3 * NBUF]
    souts = bufs_obs_sems[3 * NBUF:4 * NBUF]

    wid = lax.axis_index("s") * NUM_CORES + lax.axis_index("c")
    pltpu.sync_copy(ys_hbm, ysv)
    pltpu.sync_copy(xs_hbm, xsv)
    lane = lax.iota(jnp.int32, 16)

    def item_coords(k):
        p = k & 15
        i = k >> 4
        j = i >> 2
        b = i & 3
        mask = lane == p
        y0 = jnp.sum(jnp.where(mask, ysv[...], 0))
        x0 = jnp.sum(jnp.where(mask, xsv[...], 0))
        c = wid * C_PER_WORKER + j
        grow = (b * C + c) * IMG_H + y0
        ra = jnp.minimum(grow & ~7, N_ROWS - BLK_H)
        dy = grow - ra
        xa = jnp.minimum(x0 & ~127, IMG_W - BLK_W)
        dx = x0 - xa
        orow = ((p * B + b) * C + c) * PATCH_H
        return ra, xa, dy, dx, orow

    def start_in(k, par):
        ra, xa, _, dx, _ = item_coords(k)
        srcl = x_hbm.at[
            pl.ds(pl.multiple_of(ra, 8), BLK_H),
            pl.ds(pl.multiple_of(xa, 128), 128),
        ]
        pltpu.async_copy(srcl, bufs[par].at[:, pl.ds(0, 128)], sins[par])

        @pl.when(dx > 64)
        def _():
            srcr = x_hbm.at[
                pl.ds(pl.multiple_of(ra, 8), BLK_H),
                pl.ds(pl.multiple_of(xa + 128, 128), 128),
            ]
            pltpu.async_copy(srcr, bufs[par].at[:, pl.ds(128, 128)], sins[par])

    def wait_in(k, par):
        _, _, _, dx, _ = item_coords(k)
        pltpu.make_async_copy(
            x_hbm.at[pl.ds(0, BLK_H), pl.ds(0, 128)],
            bufs[par].at[:, pl.ds(0, 128)],
            sins[par],
        ).wait()

        @pl.when(dx > 64)
        def _():
            pltpu.make_async_copy(
                x_hbm.at[pl.ds(0, BLK_H), pl.ds(0, 128)],
                bufs[par].at[:, pl.ds(128, 128)],
                sins[par],
            ).wait()

    def start_out(orow, par):
        dst = out_hbm.at[pl.ds(pl.multiple_of(orow, 8), PATCH_H), :]
        pltpu.async_copy(obs[par], dst, souts[par])

    def wait_out(par):
        pltpu.make_async_copy(
            obs[par], out_hbm.at[pl.ds(0, PATCH_H), :], souts[par]
        ).wait()

    def extract(dy, dx, par):
        buf = bufs[par]
        ob = obs[par]
        colvs = [dx + jj * 16 + lane for jj in range(PATCH_W // 16)]

        def r_body(r, _):
            rowv = lax.full((16,), 0, jnp.int32) + (dy + r)
            for jj in range(PATCH_W // 16):
                v = plsc.load_gather(buf, [rowv, colvs[jj]])
                ob[r, pl.ds(jj * 16, 16)] = v
            return _

        lax.fori_loop(0, PATCH_H, r_body, 0)

    for par in range(NBUF):
        start_in(par, par)

    def group_body(kg, carry):
        for par in range(NBUF):
            k = NBUF * kg + par
            _, _, dy, dx, orow = item_coords(k)
            wait_in(k, par)

            @pl.when(k >= NBUF)
            def _():
                wait_out(par)

            extract(dy, dx, par)
            start_out(orow, par)

            @pl.when(k + NBUF < ITEMS)
            def _():
                start_in(k + NBUF, par)
        return carry

    lax.fori_loop(0, ITEMS // NBUF, group_body, 0)
    for par in range(NBUF):
        wait_out(par)


@jax.jit
def _run(x2, ys, xs):
    mesh = plsc.VectorSubcoreMesh(core_axis_name="c", subcore_axis_name="s")
    f = functools.partial(
        pl.kernel,
        mesh=mesh,
        out_type=jax.ShapeDtypeStruct(
            (PATCH_NUM * B * C * PATCH_H, PATCH_W), jnp.float32
        ),
        scratch_types=(
            [
                pltpu.VMEM((16,), jnp.int32),
                pltpu.VMEM((16,), jnp.int32),
            ]
            + [pltpu.VMEM((BLK_H, BLK_W), jnp.float32)] * NBUF
            + [pltpu.VMEM((PATCH_H, PATCH_W), jnp.float32)] * NBUF
            + [pltpu.SemaphoreType.DMA] * (2 * NBUF)
        ),
        compiler_params=pltpu.CompilerParams(needs_layout_passes=False),
    )(_patch_body)
    return f(x2, ys, xs)


def kernel(input, patch_indices):
    x2 = input.reshape(B * C * IMG_H, IMG_W)
    pidx = patch_indices.astype(jnp.int32)
    ys = pidx[:, 0]
    xs = pidx[:, 1]
    out2 = _run(x2, ys, xs)
    return out2.reshape(PATCH_NUM * B, C, PATCH_H, PATCH_W)


# speedup vs baseline: 1.5349x; 1.1299x over previous
import functools

import jax
import jax.numpy as jnp
from jax import lax
from jax.experimental import pallas as pl
from jax.experimental.pallas import tpu as pltpu
from jax.experimental.pallas import tpu_sc as plsc

IMG_H, IMG_W = 512, 512
PATCH_H, PATCH_W = 64, 64
PATCH_NUM = 16
B, C = 4, 96

NUM_CORES = 2
NUM_SUBCORES = 16
NUM_WORKERS = NUM_CORES * NUM_SUBCORES
C_PER_WORKER = C // NUM_WORKERS
ITEMS = PATCH_NUM * B * C_PER_WORKER
NBUF = 4

N_ROWS = B * C * IMG_H
BLK_H = PATCH_H + 8
BLK_W = 256


def _patch_body(x_hbm, ys_hbm, xs_hbm, out_hbm, ysv, xsv, *bufs_obs_sems):
    bufs = bufs_obs_sems[0:NBUF]
    obs = bufs_obs_sems[NBUF:2 * NBUF]
    sins = bufs_obs_sems[2 * NBUF:3 * NBUF]
    souts = bufs_obs_sems[3 * NBUF:4 * NBUF]

    wid = lax.axis_index("s") * NUM_CORES + lax.axis_index("c")
    pltpu.sync_copy(ys_hbm, ysv)
    pltpu.sync_copy(xs_hbm, xsv)
    lane = lax.iota(jnp.int32, 16)

    def item_coords(k):
        p = k & 15
        i = k >> 4
        j = i >> 2
        b = i & 3
        mask = lane == p
        y0 = jnp.sum(jnp.where(mask, ysv[...], 0))
        x0 = jnp.sum(jnp.where(mask, xsv[...], 0))
        c = wid * C_PER_WORKER + j
        grow = (b * C + c) * IMG_H + y0
        ra = jnp.minimum(grow & ~7, N_ROWS - BLK_H)
        dy = grow - ra
        xa = jnp.minimum(x0 & ~127, IMG_W - BLK_W)
        dx = x0 - xa
        orow = ((p * B + b) * C + c) * PATCH_H
        return ra, xa, dy, dx, orow

    def start_in(k, par):
        ra, xa, _, dx, _ = item_coords(k)
        srcl = x_hbm.at[
            pl.ds(pl.multiple_of(ra, 8), BLK_H),
            pl.ds(pl.multiple_of(xa, 128), 128),
        ]
        pltpu.async_copy(srcl, bufs[par].at[:, pl.ds(0, 128)], sins[par])

        @pl.when(dx > 64)
        def _():
            srcr = x_hbm.at[
                pl.ds(pl.multiple_of(ra, 8), BLK_H),
                pl.ds(pl.multiple_of(xa + 128, 128), 128),
            ]
            pltpu.async_copy(srcr, bufs[par].at[:, pl.ds(128, 128)], sins[par])

    def wait_in(k, par):
        _, _, _, dx, _ = item_coords(k)
        pltpu.make_async_copy(
            x_hbm.at[pl.ds(0, BLK_H), pl.ds(0, 128)],
            bufs[par].at[:, pl.ds(0, 128)],
            sins[par],
        ).wait()

        @pl.when(dx > 64)
        def _():
            pltpu.make_async_copy(
                x_hbm.at[pl.ds(0, BLK_H), pl.ds(0, 128)],
                bufs[par].at[:, pl.ds(128, 128)],
                sins[par],
            ).wait()

    def start_out(orow, par):
        dst = out_hbm.at[pl.ds(pl.multiple_of(orow, 8), PATCH_H), :]
        pltpu.async_copy(obs[par], dst, souts[par])

    def wait_out(par):
        pltpu.make_async_copy(
            obs[par], out_hbm.at[pl.ds(0, PATCH_H), :], souts[par]
        ).wait()

    def extract(dy, dx, par):
        buf = bufs[par]
        ob = obs[par]
        colvs = [dx + jj * 16 + lane for jj in range(PATCH_W // 16)]

        @plsc.parallel_loop(0, PATCH_H, 1, unroll=4)
        def _(r):
            rowv = lax.full((16,), 0, jnp.int32) + (dy + r)
            for jj in range(PATCH_W // 16):
                v = plsc.load_gather(buf, [rowv, colvs[jj]])
                ob[r, pl.ds(jj * 16, 16)] = v

    for par in range(NBUF):
        start_in(par, par)

    def group_body(kg, carry):
        for par in range(NBUF):
            k = NBUF * kg + par
            _, _, dy, dx, orow = item_coords(k)
            wait_in(k, par)

            @pl.when(k >= NBUF)
            def _():
                wait_out(par)

            extract(dy, dx, par)
            start_out(orow, par)

            @pl.when(k + NBUF < ITEMS)
            def _():
                start_in(k + NBUF, par)
        return carry

    lax.fori_loop(0, ITEMS // NBUF, group_body, 0)
    for par in range(NBUF):
        wait_out(par)


@jax.jit
def _run(x2, ys, xs):
    mesh = plsc.VectorSubcoreMesh(core_axis_name="c", subcore_axis_name="s")
    f = functools.partial(
        pl.kernel,
        mesh=mesh,
        out_type=jax.ShapeDtypeStruct(
            (PATCH_NUM * B * C * PATCH_H, PATCH_W), jnp.float32
        ),
        scratch_types=(
            [
                pltpu.VMEM((16,), jnp.int32),
                pltpu.VMEM((16,), jnp.int32),
            ]
            + [pltpu.VMEM((BLK_H, BLK_W), jnp.float32)] * NBUF
            + [pltpu.VMEM((PATCH_H, PATCH_W), jnp.float32)] * NBUF
            + [pltpu.SemaphoreType.DMA] * (2 * NBUF)
        ),
        compiler_params=pltpu.CompilerParams(needs_layout_passes=False),
    )(_patch_body)
    return f(x2, ys, xs)


def kernel(input, patch_indices):
    x2 = input.reshape(B * C * IMG_H, IMG_W)
    pidx = patch_indices.astype(jnp.int32)
    ys = pidx[:, 0]
    xs = pidx[:, 1]
    out2 = _run(x2, ys, xs)
    return out2.reshape(PATCH_NUM * B, C, PATCH_H, PATCH_W)


# speedup vs baseline: 1.5353x; 1.0003x over previous
import functools

import jax
import jax.numpy as jnp
from jax import lax
from jax.experimental import pallas as pl
from jax.experimental.pallas import tpu as pltpu
from jax.experimental.pallas import tpu_sc as plsc

IMG_H, IMG_W = 512, 512
PATCH_H, PATCH_W = 64, 64
PATCH_NUM = 16
B, C = 4, 96

NUM_CORES = 2
NUM_SUBCORES = 16
NUM_WORKERS = NUM_CORES * NUM_SUBCORES
C_PER_WORKER = C // NUM_WORKERS
ITEMS = PATCH_NUM * B * C_PER_WORKER
NBUF = 4

N_ROWS = B * C * IMG_H
BLK_H = PATCH_H + 8
BLK_W = 256


def _patch_body(x_hbm, ys_hbm, xs_hbm, out_hbm, ysv, xsv, *bufs_obs_sems):
    bufs = bufs_obs_sems[0:NBUF]
    obs = bufs_obs_sems[NBUF:2 * NBUF]
    sins = bufs_obs_sems[2 * NBUF:3 * NBUF]
    souts = bufs_obs_sems[3 * NBUF:4 * NBUF]

    wid = lax.axis_index("s") * NUM_CORES + lax.axis_index("c")
    pltpu.sync_copy(ys_hbm, ysv)
    pltpu.sync_copy(xs_hbm, xsv)
    lane = lax.iota(jnp.int32, 16)

    def item_coords(k):
        p = k & 15
        i = k >> 4
        j = i >> 2
        b = i & 3
        mask = lane == p
        y0 = jnp.sum(jnp.where(mask, ysv[...], 0))
        x0 = jnp.sum(jnp.where(mask, xsv[...], 0))
        c = wid * C_PER_WORKER + j
        grow = (b * C + c) * IMG_H + y0
        ra = jnp.minimum(grow & ~7, N_ROWS - BLK_H)
        dy = grow - ra
        xa = jnp.minimum(x0 & ~127, IMG_W - BLK_W)
        dx = x0 - xa
        orow = ((p * B + b) * C + c) * PATCH_H
        return ra, xa, dy, dx, orow

    def start_in(k, par):
        ra, xa, _, dx, _ = item_coords(k)
        srcl = x_hbm.at[
            pl.ds(pl.multiple_of(ra, 8), BLK_H),
            pl.ds(pl.multiple_of(xa, 128), 128),
        ]
        pltpu.async_copy(srcl, bufs[par].at[:, pl.ds(0, 128)], sins[par])

        @pl.when(dx > 64)
        def _():
            srcr = x_hbm.at[
                pl.ds(pl.multiple_of(ra, 8), BLK_H),
                pl.ds(pl.multiple_of(xa + 128, 128), 128),
            ]
            pltpu.async_copy(srcr, bufs[par].at[:, pl.ds(128, 128)], sins[par])

    def wait_in(k, par):
        _, _, _, dx, _ = item_coords(k)
        pltpu.make_async_copy(
            x_hbm.at[pl.ds(0, BLK_H), pl.ds(0, 128)],
            bufs[par].at[:, pl.ds(0, 128)],
            sins[par],
        ).wait()

        @pl.when(dx > 64)
        def _():
            pltpu.make_async_copy(
                x_hbm.at[pl.ds(0, BLK_H), pl.ds(0, 128)],
                bufs[par].at[:, pl.ds(128, 128)],
                sins[par],
            ).wait()

    def start_out(orow, par):
        dst = out_hbm.at[pl.ds(pl.multiple_of(orow, 8), PATCH_H), :]
        pltpu.async_copy(obs[par], dst, souts[par])

    def wait_out(par):
        pltpu.make_async_copy(
            obs[par], out_hbm.at[pl.ds(0, PATCH_H), :], souts[par]
        ).wait()

    def extract(dy, dx, par):
        buf = bufs[par]
        ob = obs[par]
        colvs = [dx + jj * 16 + lane for jj in range(PATCH_W // 16)]

        @plsc.parallel_loop(0, PATCH_H, 1, unroll=8)
        def _(r):
            rowv = lax.full((16,), 0, jnp.int32) + (dy + r)
            for jj in range(PATCH_W // 16):
                v = plsc.load_gather(buf, [rowv, colvs[jj]])
                ob[r, pl.ds(jj * 16, 16)] = v

    for par in range(NBUF):
        start_in(par, par)

    def group_body(kg, carry):
        for par in range(NBUF):
            k = NBUF * kg + par
            _, _, dy, dx, orow = item_coords(k)
            wait_in(k, par)

            @pl.when(k >= NBUF)
            def _():
                wait_out(par)

            extract(dy, dx, par)
            start_out(orow, par)

            @pl.when(k + NBUF < ITEMS)
            def _():
                start_in(k + NBUF, par)
        return carry

    lax.fori_loop(0, ITEMS // NBUF, group_body, 0)
    for par in range(NBUF):
        wait_out(par)


@jax.jit
def _run(x2, ys, xs):
    mesh = plsc.VectorSubcoreMesh(core_axis_name="c", subcore_axis_name="s")
    f = functools.partial(
        pl.kernel,
        mesh=mesh,
        out_type=jax.ShapeDtypeStruct(
            (PATCH_NUM * B * C * PATCH_H, PATCH_W), jnp.float32
        ),
        scratch_types=(
            [
                pltpu.VMEM((16,), jnp.int32),
                pltpu.VMEM((16,), jnp.int32),
            ]
            + [pltpu.VMEM((BLK_H, BLK_W), jnp.float32)] * NBUF
            + [pltpu.VMEM((PATCH_H, PATCH_W), jnp.float32)] * NBUF
            + [pltpu.SemaphoreType.DMA] * (2 * NBUF)
        ),
        compiler_params=pltpu.CompilerParams(needs_layout_passes=False),
    )(_patch_body)
    return f(x2, ys, xs)


def kernel(input, patch_indices):
    x2 = input.reshape(B * C * IMG_H, IMG_W)
    pidx = patch_indices.astype(jnp.int32)
    ys = pidx[:, 0]
    xs = pidx[:, 1]
    out2 = _run(x2, ys, xs)
    return out2.reshape(PATCH_NUM * B, C, PATCH_H, PATCH_W)


# speedup vs baseline: 1.5366x; 1.0009x over previous
import functools

import jax
import jax.numpy as jnp
from jax import lax
from jax.experimental import pallas as pl
from jax.experimental.pallas import tpu as pltpu
from jax.experimental.pallas import tpu_sc as plsc

IMG_H, IMG_W = 512, 512
PATCH_H, PATCH_W = 64, 64
PATCH_NUM = 16
B, C = 4, 96

NUM_CORES = 2
NUM_SUBCORES = 16
NUM_WORKERS = NUM_CORES * NUM_SUBCORES
C_PER_WORKER = C // NUM_WORKERS
ITEMS = PATCH_NUM * B * C_PER_WORKER
NBUF = 4

N_ROWS = B * C * IMG_H
BLK_H = PATCH_H + 8
BLK_W = 256


def _patch_body(x_hbm, ys_hbm, xs_hbm, out_hbm, ysv, xsv, *bufs_obs_sems):
    bufs = bufs_obs_sems[0:NBUF]
    obs = bufs_obs_sems[NBUF:2 * NBUF]
    sins = bufs_obs_sems[2 * NBUF:3 * NBUF]
    souts = bufs_obs_sems[3 * NBUF:4 * NBUF]

    wid = lax.axis_index("s") * NUM_CORES + lax.axis_index("c")
    pltpu.sync_copy(ys_hbm, ysv)
    pltpu.sync_copy(xs_hbm, xsv)
    lane = lax.iota(jnp.int32, 16)

    def item_coords(k):
        p = k & 15
        i = k >> 4
        j = i >> 2
        b = i & 3
        mask = lane == p
        y0 = jnp.sum(jnp.where(mask, ysv[...], 0))
        x0 = jnp.sum(jnp.where(mask, xsv[...], 0))
        c = wid * C_PER_WORKER + j
        grow = (b * C + c) * IMG_H + y0
        ra = jnp.minimum(grow & ~7, N_ROWS - BLK_H)
        dy = grow - ra
        xa = jnp.minimum(x0 & ~127, IMG_W - BLK_W)
        dx = x0 - xa
        oidx = (p * B + b) * C + c
        return ra, xa, dy, dx, oidx

    def start_in(k, par):
        ra, xa, _, dx, _ = item_coords(k)
        srcl = x_hbm.at[
            pl.ds(pl.multiple_of(ra, 8), BLK_H),
            pl.ds(pl.multiple_of(xa, 128), 128),
        ]
        pltpu.async_copy(srcl, bufs[par].at[:, pl.ds(0, 128)], sins[par])

        @pl.when(dx > 64)
        def _():
            srcr = x_hbm.at[
                pl.ds(pl.multiple_of(ra, 8), BLK_H),
                pl.ds(pl.multiple_of(xa + 128, 128), 128),
            ]
            pltpu.async_copy(srcr, bufs[par].at[:, pl.ds(128, 128)], sins[par])

    def wait_in(k, par):
        _, _, _, dx, _ = item_coords(k)
        pltpu.make_async_copy(
            x_hbm.at[pl.ds(0, BLK_H), pl.ds(0, 128)],
            bufs[par].at[:, pl.ds(0, 128)],
            sins[par],
        ).wait()

        @pl.when(dx > 64)
        def _():
            pltpu.make_async_copy(
                x_hbm.at[pl.ds(0, BLK_H), pl.ds(0, 128)],
                bufs[par].at[:, pl.ds(128, 128)],
                sins[par],
            ).wait()

    def start_out(oidx, par):
        pltpu.async_copy(obs[par], out_hbm.at[oidx], souts[par])

    def wait_out(par):
        pltpu.make_async_copy(obs[par], out_hbm.at[0], souts[par]).wait()

    def extract(dy, dx, par):
        buf = bufs[par]
        ob = obs[par]
        colvs = [dx + jj * 16 + lane for jj in range(PATCH_W // 16)]

        @plsc.parallel_loop(0, PATCH_H, 1, unroll=8)
        def _(r):
            rowv = lax.full((16,), 0, jnp.int32) + (dy + r)
            for jj in range(PATCH_W // 16):
                v = plsc.load_gather(buf, [rowv, colvs[jj]])
                ob[r, pl.ds(jj * 16, 16)] = v

    for par in range(NBUF):
        start_in(par, par)

    def group_body(kg, carry):
        for par in range(NBUF):
            k = NBUF * kg + par
            _, _, dy, dx, oidx = item_coords(k)
            wait_in(k, par)

            @pl.when(k >= NBUF)
            def _():
                wait_out(par)

            extract(dy, dx, par)
            start_out(oidx, par)

            @pl.when(k + NBUF < ITEMS)
            def _():
                start_in(k + NBUF, par)
        return carry

    lax.fori_loop(0, ITEMS // NBUF, group_body, 0)
    for par in range(NBUF):
        wait_out(par)


@jax.jit
def _run(x2, ys, xs):
    mesh = plsc.VectorSubcoreMesh(core_axis_name="c", subcore_axis_name="s")
    f = functools.partial(
        pl.kernel,
        mesh=mesh,
        out_type=jax.ShapeDtypeStruct(
            (PATCH_NUM * B * C, PATCH_H, PATCH_W), jnp.float32
        ),
        scratch_types=(
            [
                pltpu.VMEM((16,), jnp.int32),
                pltpu.VMEM((16,), jnp.int32),
            ]
            + [pltpu.VMEM((BLK_H, BLK_W), jnp.float32)] * NBUF
            + [pltpu.VMEM((PATCH_H, PATCH_W), jnp.float32)] * NBUF
            + [pltpu.SemaphoreType.DMA] * (2 * NBUF)
        ),
        compiler_params=pltpu.CompilerParams(needs_layout_passes=False),
    )(_patch_body)
    return f(x2, ys, xs)


def kernel(input, patch_indices):
    x2 = input.reshape(B * C * IMG_H, IMG_W)
    pidx = patch_indices.astype(jnp.int32)
    ys = pidx[:, 0]
    xs = pidx[:, 1]
    out3 = _run(x2, ys, xs)
    return out3.reshape(PATCH_NUM * B, C, PATCH_H, PATCH_W)
